# baseline (speedup 1.0000x reference)
import functools

import jax
import jax.numpy as jnp
from jax import lax
from jax.experimental import pallas as pl
from jax.experimental.pallas import tpu as pltpu
from jax.experimental.pallas import tpu_sc as plsc

K = 40
NB = 128
RB = 256
SC_NC, SC_NS = 2, 16
SC_NW = SC_NC * SC_NS
SC_CHUNK = 128


def _lrelu(x):
    return jnp.where(x >= 0, x, 0.2 * x)


def _relu(x):
    return jnp.maximum(x, 0.0)


def _dist(x):
    B, N, C = x.shape
    xt = jnp.swapaxes(x, 1, 2)
    GB = 128

    def body(xa_ref, xt_ref, d_ref):
        xa = xa_ref[0]
        xtg = xt_ref[0]
        sqa = jnp.sum(xa * xa, axis=1, keepdims=True)
        sqg = jnp.sum(xtg * xtg, axis=0, keepdims=True)
        d = sqa + sqg - 2.0 * jnp.dot(xa, xtg, preferred_element_type=jnp.float32)
        for gl in range(GB // 16):
            d_ref[0, gl] = d[:, gl * 16:(gl + 1) * 16]

    return pl.pallas_call(
        body, grid=(B, N // GB),
        in_specs=[pl.BlockSpec((1, N, C), lambda b, g: (b, 0, 0)),
                  pl.BlockSpec((1, C, GB), lambda b, g: (b, 0, g))],
        out_specs=pl.BlockSpec((1, GB // 16, N, 16), lambda b, g: (b, g, 0, 0)),
        out_shape=jax.ShapeDtypeStruct((B, N // 16, N, 16), jnp.float32))(x, xt)


def _sc_gather(table, idx_flat):
    R, CH = table.shape
    E = idx_flat.shape[0]
    per_w = E // SC_NW
    steps = per_w // SC_CHUNK
    assert steps % 2 == 0
    mesh = plsc.VectorSubcoreMesh(core_axis_name="c", subcore_axis_name="s")

    @functools.partial(
        pl.kernel, mesh=mesh,
        compiler_params=pltpu.CompilerParams(use_tc_tiling_on_sc=False),
        out_type=jax.ShapeDtypeStruct((E, CH), jnp.float32),
        scratch_types=[pltpu.VMEM((per_w,), jnp.int32),
                       pltpu.VMEM((SC_CHUNK, CH), jnp.float32),
                       pltpu.VMEM((SC_CHUNK, CH), jnp.float32),
                       pltpu.SemaphoreType.DMA, pltpu.SemaphoreType.DMA,
                       pltpu.SemaphoreType.DMA, pltpu.SemaphoreType.DMA],
    )
    def gk(table_hbm, idx_hbm, out_hbm, idx_all, r0, r1, sg0, sg1, so0, so1):
        wid = lax.axis_index("s") * SC_NC + lax.axis_index("c")
        base = wid * per_w
        pltpu.sync_copy(idx_hbm.at[pl.ds(base, per_w)], idx_all)

        def gat(i, rows, sem):
            return pltpu.make_async_copy(
                table_hbm.at[idx_all.at[pl.ds(i * SC_CHUNK, SC_CHUNK)]],
                rows, sem)

        def out(i, rows, sem):
            off = pl.multiple_of(base + i * SC_CHUNK, SC_CHUNK)
            return pltpu.make_async_copy(
                rows, out_hbm.at[pl.ds(off, SC_CHUNK)], sem)

        def pair(p, carry):
            i0 = p * 2
            i1 = i0 + 1

            @pl.when(p > 0)
            def _():
                out(i0 - 2, r0, so0).wait()

            gat(i0, r0, sg0).start()

            @pl.when(p > 0)
            def _():
                out(i1 - 2, r1, so1).wait()

            gat(i1, r1, sg1).start()
            gat(i0, r0, sg0).wait()
            out(i0, r0, so0).start()
            gat(i1, r1, sg1).wait()
            out(i1, r1, so1).start()
            return carry

        lax.fori_loop(0, steps // 2, pair, 0)
        out(steps - 2, r0, so0).wait()
        out(steps - 1, r1, so1).wait()

    return gk(table, idx_flat)


def _gather_edges(P, idx_t):
    B, N, CH = P.shape
    flat = (idx_t + (jnp.arange(B, dtype=jnp.int32) * N).reshape(B, 1, 1))
    flat = flat.reshape(-1).astype(jnp.int32)
    G = _sc_gather(P.reshape(B * N, CH), flat)
    return G.reshape(B, K, N, CH)


def _sc_topk(DT):
    B, NG, N, _ = DT.shape
    n_groups = B * NG
    gpw = n_groups // SC_NW
    wpb = SC_NW // B
    NC1, NC2 = N // 16, N // 256
    mesh = plsc.VectorSubcoreMesh(core_axis_name="c", subcore_axis_name="s")
    INF = jnp.float32(3.0e38)
    DTf = DT.reshape(B, NG, N * 16)

    @functools.partial(
        pl.kernel, mesh=mesh,
        compiler_params=pltpu.CompilerParams(needs_layout_passes=False),
        out_type=jax.ShapeDtypeStruct((B, NG, K * 16), jnp.int32),
        scratch_types=[pltpu.VMEM((N * 16,), jnp.float32),
                       pltpu.VMEM((NC1 * 16,), jnp.float32),
                       pltpu.VMEM((NC2 * 16,), jnp.float32),
                       pltpu.VMEM((K * 16,), jnp.int32)],
    )
    def tk(d_hbm, idx_hbm, dcol, cm, m2, ibuf):
        wid = lax.axis_index("s") * SC_NC + lax.axis_index("c")
        lanes = lax.iota(jnp.int32, 16)
        b = wid // wpb
        g0 = (wid % wpb) * gpw

        def group_body(gi, carry):
            g = g0 + gi
            pltpu.sync_copy(d_hbm.at[b, g], dcol)

            def cmin_body(c, c2):
                m = dcol[pl.ds(c * 256, 16)]
                for t in range(1, 16):
                    m = jnp.minimum(m, dcol[pl.ds(c * 256 + t * 16, 16)])
                cm[pl.ds(c * 16, 16)] = m
                return c2

            lax.fori_loop(0, NC1, cmin_body, 0)

            def gmin_body(g2, c2):
                m = cm[pl.ds(g2 * 256, 16)]
                for t in range(1, 16):
                    m = jnp.minimum(m, cm[pl.ds(g2 * 256 + t * 16, 16)])
                m2[pl.ds(g2 * 16, 16)] = m
                return c2

            lax.fori_loop(0, NC2, gmin_body, 0)

            def ext_body(kk, c2):
                gmin = m2[pl.ds(0, 16)]
                for i in range(1, NC2):
                    gmin = jnp.minimum(gmin, m2[pl.ds(i * 16, 16)])
                gsel = jnp.zeros((16,), jnp.int32)
                for i in range(NC2 - 1, -1, -1):
                    gsel = jnp.where(m2[pl.ds(i * 16, 16)] == gmin,
                                     jnp.full((16,), i, jnp.int32), gsel)
                cbase = gsel * 16
                csel = jnp.zeros((16,), jnp.int32)
                for t in range(15, -1, -1):
                    val = plsc.load_gather(cm, [(cbase + t) * 16 + lanes])
                    csel = jnp.where(val == gmin, cbase + t, csel)
                jbase = csel * 16
                jsel = jnp.zeros((16,), jnp.int32)
                for t in range(15, -1, -1):
                    val = plsc.load_gather(dcol, [(jbase + t) * 16 + lanes])
                    jsel = jnp.where(val == gmin, jbase + t, jsel)
                ibuf[pl.ds(kk * 16, 16)] = jsel
                plsc.store_scatter(dcol, [jsel * 16 + lanes],
                                   jnp.full((16,), INF, jnp.float32))
                m = jnp.full((16,), INF, jnp.float32)
                for t in range(16):
                    m = jnp.minimum(m, plsc.load_gather(dcol, [(jbase + t) * 16 + lanes]))
                plsc.store_scatter(cm, [csel * 16 + lanes], m)
                m = jnp.full((16,), INF, jnp.float32)
                for t in range(16):
                    m = jnp.minimum(m, plsc.load_gather(cm, [(cbase + t) * 16 + lanes]))
                plsc.store_scatter(m2, [gsel * 16 + lanes], m)
                return c2

            lax.fori_loop(0, K, ext_body, 0)
            pltpu.sync_copy(ibuf, idx_hbm.at[b, g])
            return carry

        lax.fori_loop(0, gpw, group_body, 0)

    idx4 = tk(DTf).reshape(B, NG, K, 16)
    return jnp.transpose(idx4, (0, 2, 1, 3)).reshape(B, K, N)


def _edge_stage(G, x, W1, b1, W2, b2, res=None):
    B, Kk, N, CP = G.shape
    C = x.shape[-1]
    CO = W2.shape[1]

    def body(*refs):
        if res is None:
            g_ref, x_ref, w1_ref, b1_ref, w2_ref, b2_ref, o_ref = refs
        else:
            (g_ref, x_ref, w1_ref, b1_ref, w2_ref, b2_ref,
             wl1_ref, bl1_ref, wl2_ref, bl2_ref, wlt_ref, blt_ref, o_ref) = refs
        g = g_ref[0]
        xc = x_ref[0]
        f = g[:, :, :C]
        ctr = jnp.broadcast_to(xc[None], (Kk, NB, C))
        e = jnp.concatenate([f - ctr, ctr], axis=2).reshape(Kk * NB, 2 * C)
        h = _lrelu(jnp.dot(e, w1_ref[...], preferred_element_type=jnp.float32)
                   + b1_ref[...])
        h = _lrelu(jnp.dot(h, w2_ref[...], preferred_element_type=jnp.float32)
                   + b2_ref[...])
        hm = jnp.max(h.reshape(Kk, NB, CO), axis=0)
        if res is None:
            o_ref[0] = hm
        else:
            h1 = _relu(jnp.dot(hm, wl1_ref[...], preferred_element_type=jnp.float32)
                       + bl1_ref[...])
            h2 = (jnp.dot(h1, wl2_ref[...], preferred_element_type=jnp.float32)
                  + bl2_ref[...])
            xlt = (jnp.dot(xc, wlt_ref[...], preferred_element_type=jnp.float32)
                   + blt_ref[...])
            o_ref[0] = xlt + h2

    def _full(w):
        return pl.BlockSpec(w.shape, lambda b, n: (0,) * w.ndim)

    ins = [G, x, W1, b1.reshape(1, -1), W2, b2.reshape(1, -1)]
    in_specs = [pl.BlockSpec((1, Kk, NB, CP), lambda b, n: (b, 0, n, 0)),
                pl.BlockSpec((1, NB, C), lambda b, n: (b, n, 0)),
                _full(W1), _full(b1.reshape(1, -1)),
                _full(W2), _full(b2.reshape(1, -1))]
    if res is not None:
        Wl1, bl1, Wl2, bl2, Wlt, blt = res
        extra = [Wl1, bl1.reshape(1, -1), Wl2, bl2.reshape(1, -1),
                 Wlt, blt.reshape(1, -1)]
        ins += extra
        in_specs += [_full(a) for a in extra]
    return pl.pallas_call(
        body, grid=(B, N // NB), in_specs=in_specs,
        out_specs=pl.BlockSpec((1, NB, CO), lambda b, n: (b, n, 0)),
        out_shape=jax.ShapeDtypeStruct((B, N, CO), jnp.float32))(*ins)


def _t2_globalmax(x, W, b):
    B, N, C = x.shape
    CO = W.shape[1]

    def body(x_ref, w_ref, b_ref, o_ref):
        nb = pl.program_id(1)
        y = _lrelu(jnp.dot(x_ref[0], w_ref[...], preferred_element_type=jnp.float32)
                   + b_ref[...])
        m = jnp.max(y, axis=0, keepdims=True)
        o_ref[0] = jnp.where(nb == 0, m, jnp.maximum(o_ref[0], m))

    out = pl.pallas_call(
        body, grid=(B, N // RB),
        in_specs=[pl.BlockSpec((1, RB, C), lambda b, n: (b, n, 0)),
                  pl.BlockSpec(W.shape, lambda b, n: (0, 0)),
                  pl.BlockSpec((1, CO), lambda b, n: (0, 0))],
        out_specs=pl.BlockSpec((1, 1, CO), lambda b, n: (b, 0, 0)),
        out_shape=jax.ShapeDtypeStruct((B, 1, CO), jnp.float32))(x, W, b.reshape(1, CO))
    return out.reshape(B, CO)


def _tnet_tail(g, t3, t4):
    (W3a, b3a), (W3b, b3b) = t3
    W4, b4 = t4
    B = g.shape[0]

    def body(g_ref, w3a, b3a_, w3b, b3b_, w4, b4_, o_ref):
        h = _lrelu(jnp.dot(g_ref[...], w3a[...], preferred_element_type=jnp.float32)
                   + b3a_[...])
        h = _lrelu(jnp.dot(h, w3b[...], preferred_element_type=jnp.float32)
                   + b3b_[...])
        o_ref[...] = (jnp.dot(h, w4[...], preferred_element_type=jnp.float32)
                      + b4_[...])

    return pl.pallas_call(
        body,
        out_shape=jax.ShapeDtypeStruct((B, 9), jnp.float32),
    )(g, W3a, b3a.reshape(1, -1), W3b, b3b.reshape(1, -1), W4, b4.reshape(1, -1))


def _combine(pos, T, feats):
    B, N, _ = pos.shape
    F = feats.shape[-1]

    def body(p_ref, t_ref, f_ref, o_ref):
        x0 = jnp.dot(p_ref[0], t_ref[0], preferred_element_type=jnp.float32)
        o_ref[0] = jnp.concatenate([x0, f_ref[0]], axis=1)

    return pl.pallas_call(
        body, grid=(B, N // RB),
        in_specs=[pl.BlockSpec((1, RB, 3), lambda b, n: (b, n, 0)),
                  pl.BlockSpec((1, 3, 3), lambda b, n: (b, 0, 0)),
                  pl.BlockSpec((1, RB, F), lambda b, n: (b, n, 0))],
        out_specs=pl.BlockSpec((1, RB, 3 + F), lambda b, n: (b, n, 0)),
        out_shape=jax.ShapeDtypeStruct((B, N, 3 + F), jnp.float32))(pos, T, feats)


def _knn_idx(D):
    return lax.top_k(-D, K)[1]


def _forward_one(positions, features, params):
    B, N, _ = positions.shape

    def _pad128(a):
        C = a.shape[-1]
        CP = 16 if C <= 16 else 64 if C <= 64 else 128
        return a if C == CP else jnp.pad(a, ((0, 0), (0, 0), (0, CP - C)))

    W1, b1 = params['t1'][0]
    W2, b2 = params['t1'][1]
    D = _dist(positions)
    idx = _sc_topk(D)
    G = _gather_edges(_pad128(positions), idx)
    out1 = _edge_stage(G, positions, W1, b1, W2, b2)
    g = _t2_globalmax(out1, *params['t2'][0])
    T = _tnet_tail(g, params['t3'], params['t4'])
    x = _combine(positions, T.reshape(B, 3, 3), features)

    for i in range(3):
        (Wc1, bc1), (Wc2, bc2) = params['conv'][i]
        (Wl1, bl1), (Wl2, bl2) = params['lin'][i]
        Wlt, blt = params['lt'][i]
        D = _dist(x)
        idx = _sc_topk(D)
        G = _gather_edges(_pad128(x), idx)
        x = _edge_stage(G, x, Wc1, bc1, Wc2, bc2,
                        res=(Wl1, bl1, Wl2, bl2, Wlt, blt))
    return x


def kernel(positions, features, params):
    return _forward_one(positions, features, params)


# speedup vs baseline: 1.0225x; 1.0225x over previous
import functools

import jax
import jax.numpy as jnp
from jax import lax
from jax.experimental import pallas as pl
from jax.experimental.pallas import tpu as pltpu
from jax.experimental.pallas import tpu_sc as plsc

K = 40
NB = 128
RB = 256
SC_NC, SC_NS = 2, 16
SC_NW = SC_NC * SC_NS
SC_CHUNK = 128


def _lrelu(x):
    return jnp.where(x >= 0, x, 0.2 * x)


def _relu(x):
    return jnp.maximum(x, 0.0)


def _dist(x):
    B, N, C = x.shape
    xt = jnp.swapaxes(x, 1, 2)
    GB = 128

    def body(xa_ref, xt_ref, d_ref):
        xa = xa_ref[0]
        xtg = xt_ref[0]
        sqa = jnp.sum(xa * xa, axis=1, keepdims=True)
        sqg = jnp.sum(xtg * xtg, axis=0, keepdims=True)
        d = sqa + sqg - 2.0 * jnp.dot(xa, xtg, preferred_element_type=jnp.float32)
        for gl in range(GB // 16):
            d_ref[0, gl] = d[:, gl * 16:(gl + 1) * 16]

    return pl.pallas_call(
        body, grid=(B, N // GB),
        in_specs=[pl.BlockSpec((1, N, C), lambda b, g: (b, 0, 0)),
                  pl.BlockSpec((1, C, GB), lambda b, g: (b, 0, g))],
        out_specs=pl.BlockSpec((1, GB // 16, N, 16), lambda b, g: (b, g, 0, 0)),
        out_shape=jax.ShapeDtypeStruct((B, N // 16, N, 16), jnp.float32))(x, xt)


def _sc_gather(table, idx_flat):
    R, CH = table.shape
    E = idx_flat.shape[0]
    per_w = E // SC_NW
    steps = per_w // SC_CHUNK
    assert steps % 2 == 0
    mesh = plsc.VectorSubcoreMesh(core_axis_name="c", subcore_axis_name="s")

    @functools.partial(
        pl.kernel, mesh=mesh,
        compiler_params=pltpu.CompilerParams(use_tc_tiling_on_sc=False),
        out_type=jax.ShapeDtypeStruct((E, CH), jnp.float32),
        scratch_types=[pltpu.VMEM((per_w,), jnp.int32),
                       pltpu.VMEM((SC_CHUNK, CH), jnp.float32),
                       pltpu.VMEM((SC_CHUNK, CH), jnp.float32),
                       pltpu.SemaphoreType.DMA, pltpu.SemaphoreType.DMA,
                       pltpu.SemaphoreType.DMA, pltpu.SemaphoreType.DMA],
    )
    def gk(table_hbm, idx_hbm, out_hbm, idx_all, r0, r1, sg0, sg1, so0, so1):
        wid = lax.axis_index("s") * SC_NC + lax.axis_index("c")
        base = wid * per_w
        pltpu.sync_copy(idx_hbm.at[pl.ds(base, per_w)], idx_all)

        def gat(i, rows, sem):
            return pltpu.make_async_copy(
                table_hbm.at[idx_all.at[pl.ds(i * SC_CHUNK, SC_CHUNK)]],
                rows, sem)

        def out(i, rows, sem):
            off = pl.multiple_of(base + i * SC_CHUNK, SC_CHUNK)
            return pltpu.make_async_copy(
                rows, out_hbm.at[pl.ds(off, SC_CHUNK)], sem)

        def pair(p, carry):
            i0 = p * 2
            i1 = i0 + 1

            @pl.when(p > 0)
            def _():
                out(i0 - 2, r0, so0).wait()

            gat(i0, r0, sg0).start()

            @pl.when(p > 0)
            def _():
                out(i1 - 2, r1, so1).wait()

            gat(i1, r1, sg1).start()
            gat(i0, r0, sg0).wait()
            out(i0, r0, so0).start()
            gat(i1, r1, sg1).wait()
            out(i1, r1, so1).start()
            return carry

        lax.fori_loop(0, steps // 2, pair, 0)
        out(steps - 2, r0, so0).wait()
        out(steps - 1, r1, so1).wait()

    return gk(table, idx_flat)


def _gather_edges(P, idx_t):
    B, N, CH = P.shape
    flat = (idx_t + (jnp.arange(B, dtype=jnp.int32) * N).reshape(B, 1, 1))
    flat = flat.reshape(-1).astype(jnp.int32)
    G = _sc_gather(P.reshape(B * N, CH), flat)
    return G.reshape(B, K, N, CH)


def _sc_topk(DT):
    B, NG, N, _ = DT.shape
    n_groups = B * NG
    gpw = n_groups // SC_NW
    wpb = SC_NW // B
    NC1, NC2 = N // 16, N // 256
    mesh = plsc.VectorSubcoreMesh(core_axis_name="c", subcore_axis_name="s")
    INF = jnp.float32(3.0e38)
    DTf = DT.reshape(B, NG, N * 16)

    @functools.partial(
        pl.kernel, mesh=mesh,
        compiler_params=pltpu.CompilerParams(needs_layout_passes=False),
        out_type=jax.ShapeDtypeStruct((B, NG, K * 16), jnp.int32),
        scratch_types=[pltpu.VMEM((N * 16,), jnp.float32),
                       pltpu.VMEM((NC1 * 16,), jnp.float32),
                       pltpu.VMEM((NC2 * 16,), jnp.float32),
                       pltpu.VMEM((K * 16,), jnp.int32)],
    )
    def tk(d_hbm, idx_hbm, dcol, cm, m2, ibuf):
        wid = lax.axis_index("s") * SC_NC + lax.axis_index("c")
        lanes = lax.iota(jnp.int32, 16)
        b = wid // wpb
        g0 = (wid % wpb) * gpw

        def group_body(gi, carry):
            g = g0 + gi
            pltpu.sync_copy(d_hbm.at[b, g], dcol)

            def cmin_body(c, c2):
                m = dcol[pl.ds(c * 256, 16)]
                for t in range(1, 16):
                    m = jnp.minimum(m, dcol[pl.ds(c * 256 + t * 16, 16)])
                cm[pl.ds(c * 16, 16)] = m
                return c2

            lax.fori_loop(0, NC1, cmin_body, 0)

            def gmin_body(g2, c2):
                m = cm[pl.ds(g2 * 256, 16)]
                for t in range(1, 16):
                    m = jnp.minimum(m, cm[pl.ds(g2 * 256 + t * 16, 16)])
                m2[pl.ds(g2 * 16, 16)] = m
                return c2

            lax.fori_loop(0, NC2, gmin_body, 0)

            def ext_body(kk, c2):
                gmin = m2[pl.ds(0, 16)]
                for i in range(1, NC2):
                    gmin = jnp.minimum(gmin, m2[pl.ds(i * 16, 16)])
                gsel = jnp.zeros((16,), jnp.int32)
                for i in range(NC2 - 1, -1, -1):
                    gsel = jnp.where(m2[pl.ds(i * 16, 16)] == gmin,
                                     jnp.full((16,), i, jnp.int32), gsel)
                cbase = gsel * 16
                csel = jnp.zeros((16,), jnp.int32)
                for t in range(15, -1, -1):
                    val = plsc.load_gather(cm, [(cbase + t) * 16 + lanes])
                    csel = jnp.where(val == gmin, cbase + t, csel)
                jbase = csel * 16
                jsel = jnp.zeros((16,), jnp.int32)
                for t in range(15, -1, -1):
                    val = plsc.load_gather(dcol, [(jbase + t) * 16 + lanes])
                    jsel = jnp.where(val == gmin, jbase + t, jsel)
                ibuf[pl.ds(kk * 16, 16)] = jsel
                plsc.store_scatter(dcol, [jsel * 16 + lanes],
                                   jnp.full((16,), INF, jnp.float32))
                m = jnp.full((16,), INF, jnp.float32)
                for t in range(16):
                    m = jnp.minimum(m, plsc.load_gather(dcol, [(jbase + t) * 16 + lanes]))
                plsc.store_scatter(cm, [csel * 16 + lanes], m)
                m = jnp.full((16,), INF, jnp.float32)
                for t in range(16):
                    m = jnp.minimum(m, plsc.load_gather(cm, [(cbase + t) * 16 + lanes]))
                plsc.store_scatter(m2, [gsel * 16 + lanes], m)
                return c2

            lax.fori_loop(0, K, ext_body, 0)
            pltpu.sync_copy(ibuf, idx_hbm.at[b, g])
            return carry

        lax.fori_loop(0, gpw, group_body, 0)

    idx4 = tk(DTf).reshape(B, NG, K, 16)
    return jnp.transpose(idx4, (0, 2, 1, 3)).reshape(B, K, N)


def _edge_stage(G, x, W1, b1, W2, b2, res=None):
    B, Kk, N, CP = G.shape
    C = x.shape[-1]
    CO = W2.shape[1]

    def body(*refs):
        if res is None:
            g_ref, x_ref, w1_ref, b1_ref, w2_ref, b2_ref, o_ref = refs
        else:
            (g_ref, x_ref, w1_ref, b1_ref, w2_ref, b2_ref,
             wl1_ref, bl1_ref, wl2_ref, bl2_ref, wlt_ref, blt_ref, o_ref) = refs
        g = g_ref[0]
        xc = x_ref[0]
        f = g[:, :, :C]
        ctr = jnp.broadcast_to(xc[None], (Kk, NB, C))
        e = jnp.concatenate([f - ctr, ctr], axis=2).reshape(Kk * NB, 2 * C)
        h = _lrelu(jnp.dot(e, w1_ref[...], preferred_element_type=jnp.float32)
                   + b1_ref[...])
        h = _lrelu(jnp.dot(h, w2_ref[...], preferred_element_type=jnp.float32)
                   + b2_ref[...])
        hm = jnp.max(h.reshape(Kk, NB, CO), axis=0)
        if res is None:
            o_ref[0] = hm
        else:
            h1 = _relu(jnp.dot(hm, wl1_ref[...], preferred_element_type=jnp.float32)
                       + bl1_ref[...])
            h2 = (jnp.dot(h1, wl2_ref[...], preferred_element_type=jnp.float32)
                  + bl2_ref[...])
            xlt = (jnp.dot(xc, wlt_ref[...], preferred_element_type=jnp.float32)
                   + blt_ref[...])
            o_ref[0] = xlt + h2

    def _full(w):
        return pl.BlockSpec(w.shape, lambda b, n: (0,) * w.ndim)

    ins = [G, x, W1, b1.reshape(1, -1), W2, b2.reshape(1, -1)]
    in_specs = [pl.BlockSpec((1, Kk, NB, CP), lambda b, n: (b, 0, n, 0)),
                pl.BlockSpec((1, NB, C), lambda b, n: (b, n, 0)),
                _full(W1), _full(b1.reshape(1, -1)),
                _full(W2), _full(b2.reshape(1, -1))]
    if res is not None:
        Wl1, bl1, Wl2, bl2, Wlt, blt = res
        extra = [Wl1, bl1.reshape(1, -1), Wl2, bl2.reshape(1, -1),
                 Wlt, blt.reshape(1, -1)]
        ins += extra
        in_specs += [_full(a) for a in extra]
    return pl.pallas_call(
        body, grid=(B, N // NB), in_specs=in_specs,
        out_specs=pl.BlockSpec((1, NB, CO), lambda b, n: (b, n, 0)),
        out_shape=jax.ShapeDtypeStruct((B, N, CO), jnp.float32))(*ins)


def _t2_globalmax(x, W, b):
    B, N, C = x.shape
    CO = W.shape[1]

    def body(x_ref, w_ref, b_ref, o_ref):
        nb = pl.program_id(1)
        y = _lrelu(jnp.dot(x_ref[0], w_ref[...], preferred_element_type=jnp.float32)
                   + b_ref[...])
        m = jnp.max(y, axis=0, keepdims=True)
        o_ref[0] = jnp.where(nb == 0, m, jnp.maximum(o_ref[0], m))

    out = pl.pallas_call(
        body, grid=(B, N // RB),
        in_specs=[pl.BlockSpec((1, RB, C), lambda b, n: (b, n, 0)),
                  pl.BlockSpec(W.shape, lambda b, n: (0, 0)),
                  pl.BlockSpec((1, CO), lambda b, n: (0, 0))],
        out_specs=pl.BlockSpec((1, 1, CO), lambda b, n: (b, 0, 0)),
        out_shape=jax.ShapeDtypeStruct((B, 1, CO), jnp.float32))(x, W, b.reshape(1, CO))
    return out.reshape(B, CO)


def _tnet_tail(g, t3, t4):
    (W3a, b3a), (W3b, b3b) = t3
    W4, b4 = t4
    B = g.shape[0]

    def body(g_ref, w3a, b3a_, w3b, b3b_, w4, b4_, o_ref):
        h = _lrelu(jnp.dot(g_ref[...], w3a[...], preferred_element_type=jnp.float32)
                   + b3a_[...])
        h = _lrelu(jnp.dot(h, w3b[...], preferred_element_type=jnp.float32)
                   + b3b_[...])
        o_ref[...] = (jnp.dot(h, w4[...], preferred_element_type=jnp.float32)
                      + b4_[...])

    return pl.pallas_call(
        body,
        out_shape=jax.ShapeDtypeStruct((B, 9), jnp.float32),
    )(g, W3a, b3a.reshape(1, -1), W3b, b3b.reshape(1, -1), W4, b4.reshape(1, -1))


def _combine(pos, T, feats):
    B, N, _ = pos.shape
    F = feats.shape[-1]

    def body(p_ref, t_ref, f_ref, o_ref):
        x0 = jnp.dot(p_ref[0], t_ref[0], preferred_element_type=jnp.float32)
        o_ref[0] = jnp.concatenate([x0, f_ref[0]], axis=1)

    return pl.pallas_call(
        body, grid=(B, N // RB),
        in_specs=[pl.BlockSpec((1, RB, 3), lambda b, n: (b, n, 0)),
                  pl.BlockSpec((1, 3, 3), lambda b, n: (b, 0, 0)),
                  pl.BlockSpec((1, RB, F), lambda b, n: (b, n, 0))],
        out_specs=pl.BlockSpec((1, RB, 3 + F), lambda b, n: (b, n, 0)),
        out_shape=jax.ShapeDtypeStruct((B, N, 3 + F), jnp.float32))(pos, T, feats)


def _knn_idx(D):
    return lax.top_k(-D, K)[1]


def _forward_one(positions, features, params):
    B, N, _ = positions.shape

    def _pad128(a):
        C = a.shape[-1]
        CP = 16 if C <= 16 else 64 if C <= 64 else 128
        return a if C == CP else jnp.pad(a, ((0, 0), (0, 0), (0, CP - C)))

    W1, b1 = params['t1'][0]
    W2, b2 = params['t1'][1]
    D = _dist(positions)
    idx = _sc_topk(D)
    G = _gather_edges(_pad128(positions), idx)
    out1 = _edge_stage(G, positions, W1, b1, W2, b2)
    g = _t2_globalmax(out1, *params['t2'][0])
    T = _tnet_tail(g, params['t3'], params['t4'])
    x = _combine(positions, T.reshape(B, 3, 3), features)

    for i in range(3):
        (Wc1, bc1), (Wc2, bc2) = params['conv'][i]
        (Wl1, bl1), (Wl2, bl2) = params['lin'][i]
        Wlt, blt = params['lt'][i]
        D = _dist(x)
        idx = _sc_topk(D)
        G = _gather_edges(_pad128(x), idx)
        x = _edge_stage(G, x, Wc1, bc1, Wc2, bc2,
                        res=(Wl1, bl1, Wl2, bl2, Wlt, blt))
    return x


def kernel(positions, features, params):
    B = positions.shape[0]
    outs = [_forward_one(positions[b:b + 1], features[b:b + 1], params)
            for b in range(B)]
    return jnp.concatenate(outs, axis=0)
